# Initial kernel scaffold; baseline (speedup 1.0000x reference)
import functools

import jax
import jax.numpy as jnp
from jax import lax
from jax.experimental import pallas as pl
from jax.experimental.pallas import tpu as pltpu
from jax.experimental.pallas import tpu_sc as plsc

N = 10000
E = 320000
F_IN = 128
H = 128
C = 40

NC = 2
NS = 16
NW = NC * NS
NP = 10240
RPT = NP // NS
CHUNK = 128
CH_PER_W = 79
E_PAD = NW * CH_PER_W * CHUNK
ZR = 32

_f32 = jnp.float32


def _scatter_kernel_body(with_deg, D, srcs, dsts, table, *rest):
    if with_deg:
        acc_out, deg_out, s_idx, d_idx, rows, zbuf, ones_v, zdeg, acc_sh, deg_sh, sem = rest
    else:
        acc_out, s_idx, d_idx, rows, zbuf, acc_sh, sem = rest
    c = lax.axis_index("c")
    s = lax.axis_index("s")
    w = s * NC + c

    z16 = jnp.zeros((16,), _f32)
    for i in range(ZR):
        for j in range(D // 16):
            zbuf[i, pl.ds(j * 16, 16)] = z16

    @pl.loop(0, RPT // ZR)
    def _zero(i):
        pltpu.sync_copy(zbuf, acc_sh.at[pl.ds(s * RPT + i * ZR, ZR)])

    if with_deg:
        o16 = jnp.ones((16,), _f32)
        for j in range(CHUNK // 16):
            ones_v[pl.ds(j * 16, 16)] = o16
        for i in range(RPT // 16):
            zdeg[pl.ds(i * 16, 16)] = z16
        pltpu.sync_copy(zdeg, deg_sh.at[pl.ds(s * RPT, RPT)])

    base = w * CH_PER_W
    pltpu.sync_copy(srcs.at[pl.ds(base, CH_PER_W)], s_idx)
    pltpu.sync_copy(dsts.at[pl.ds(base, CH_PER_W)], d_idx)

    plsc.subcore_barrier()

    @pl.loop(0, CH_PER_W)
    def _chunk(j):
        pltpu.async_copy(table.at[s_idx.at[j]], rows, sem).wait()
        pltpu.sync_copy(rows, acc_sh.at[d_idx.at[j]], add=True)
        if with_deg:
            pltpu.sync_copy(ones_v, deg_sh.at[d_idx.at[j]], add=True)

    plsc.subcore_barrier()

    pltpu.sync_copy(acc_sh.at[pl.ds(s * RPT, RPT)],
                    acc_out.at[c, pl.ds(s * RPT, RPT)])
    if with_deg:
        pltpu.sync_copy(deg_sh.at[pl.ds(s * RPT, RPT)],
                        deg_out.at[c, pl.ds(s * RPT, RPT)])


def _make_scatter(D, with_deg):
    mesh = plsc.VectorSubcoreMesh(core_axis_name="c", subcore_axis_name="s",
                                  num_cores=NC, num_subcores=NS)
    out_type = [jax.ShapeDtypeStruct((NC, NP, D), _f32)]
    scratch = [
        pltpu.VMEM((CH_PER_W, CHUNK), jnp.int32),
        pltpu.VMEM((CH_PER_W, CHUNK), jnp.int32),
        pltpu.VMEM((CHUNK, D), _f32),
        pltpu.VMEM((ZR, D), _f32),
    ]
    if with_deg:
        out_type.append(jax.ShapeDtypeStruct((NC, NP), _f32))
        scratch += [pltpu.VMEM((CHUNK,), _f32),
                    pltpu.VMEM((RPT,), _f32)]
    scratch.append(pltpu.VMEM_SHARED((NP, D), _f32))
    if with_deg:
        scratch.append(pltpu.VMEM_SHARED((NP,), _f32))
    scratch.append(pltpu.SemaphoreType.DMA)
    return pl.kernel(
        functools.partial(_scatter_kernel_body, with_deg, D),
        out_type=out_type, mesh=mesh, scratch_types=scratch)


def _dense0_body(acc_ref, deg_ref, x_ref, wl0_ref, wr0_ref, b0_ref,
                 wl1_ref, wr1_ref, b1_ref, hw1_ref, hr1_ref, rec_ref):
    acc = acc_ref[0] + acc_ref[1]
    deg = deg_ref[0] + deg_ref[1]
    rec = 1.0 / jnp.maximum(deg, 1.0)
    agg = acc * rec
    h = agg @ wl0_ref[...] + b0_ref[...] + x_ref[...] @ wr0_ref[...]
    h = jnp.maximum(h, 0.0)
    hw1_ref[...] = h @ wl1_ref[...]
    hr1_ref[...] = h @ wr1_ref[...] + b1_ref[...]
    rec_ref[...] = rec


def _dense1_body(acc_ref, rec_ref, hr1_ref, out_ref):
    out_ref[...] = (acc_ref[0] + acc_ref[1]) * rec_ref[...] + hr1_ref[...]


_BLK = 512
_GRID = NP // _BLK


def _rep(shape):
    return pl.BlockSpec(shape, lambda i: (0,) * len(shape))


def kernel(x, edge_index, W_l0, W_r0, b0, W_l1, W_r1, b1):
    src = edge_index[0]
    dst = edge_index[1]
    npad = E_PAD - E
    pad_src = (jnp.arange(npad, dtype=jnp.int32) % N)
    pad_dst = N + (jnp.arange(npad, dtype=jnp.int32) % (NP - N))
    srcs = jnp.concatenate([src, pad_src]).reshape(E_PAD // CHUNK, CHUNK)
    dsts = jnp.concatenate([dst, pad_dst]).reshape(E_PAD // CHUNK, CHUNK)

    x_pad = jnp.pad(x, ((0, NP - N), (0, 0)))
    wl1p = jnp.pad(W_l1, ((0, 0), (0, 64 - C)))
    wr1p = jnp.pad(W_r1, ((0, 0), (0, 64 - C)))
    b0r = b0.reshape(1, H)
    b1p = jnp.pad(b1, (0, 64 - C)).reshape(1, 64)

    acc0, deg = _make_scatter(H, True)(srcs, dsts, x_pad)
    degr = deg.reshape(NC, NP, 1)

    hw1, hr1, rec = pl.pallas_call(
        _dense0_body,
        grid=(_GRID,),
        in_specs=[
            pl.BlockSpec((NC, _BLK, H), lambda i: (0, i, 0)),
            pl.BlockSpec((NC, _BLK, 1), lambda i: (0, i, 0)),
            pl.BlockSpec((_BLK, F_IN), lambda i: (i, 0)),
            _rep((F_IN, H)), _rep((F_IN, H)), _rep((1, H)),
            _rep((H, 64)), _rep((H, 64)), _rep((1, 64)),
        ],
        out_specs=[
            pl.BlockSpec((_BLK, 64), lambda i: (i, 0)),
            pl.BlockSpec((_BLK, 64), lambda i: (i, 0)),
            pl.BlockSpec((_BLK, 1), lambda i: (i, 0)),
        ],
        out_shape=[
            jax.ShapeDtypeStruct((NP, 64), _f32),
            jax.ShapeDtypeStruct((NP, 64), _f32),
            jax.ShapeDtypeStruct((NP, 1), _f32),
        ],
    )(acc0, degr, x_pad, W_l0, W_r0, b0r, wl1p, wr1p, b1p)

    (acc1,) = _make_scatter(64, False)(srcs, dsts, hw1)

    out = pl.pallas_call(
        _dense1_body,
        grid=(_GRID,),
        in_specs=[
            pl.BlockSpec((NC, _BLK, 64), lambda i: (0, i, 0)),
            pl.BlockSpec((_BLK, 1), lambda i: (i, 0)),
            pl.BlockSpec((_BLK, 64), lambda i: (i, 0)),
        ],
        out_specs=pl.BlockSpec((_BLK, 64), lambda i: (i, 0)),
        out_shape=jax.ShapeDtypeStruct((NP, 64), _f32),
    )(acc1, rec, hr1)

    return out[:N, :C]


# speedup vs baseline: 8.5785x; 8.5785x over previous
import functools

import jax
import jax.numpy as jnp
from jax import lax
from jax.experimental import pallas as pl
from jax.experimental.pallas import tpu as pltpu
from jax.experimental.pallas import tpu_sc as plsc

N = 10000
E = 320000
F_IN = 128
H = 128
C = 40

NC = 2
NS = 16
NW = NC * NS
NP = 10240
RPT = NP // NS
CHUNK = 128
CH_PER_W = 80
E_PAD = NW * CH_PER_W * CHUNK
ZR = 32

_f32 = jnp.float32


def _scatter_kernel_body(with_deg, D, srcs, dsts, table, *rest):
    if with_deg:
        acc_out, deg_out, s_idx, d_idx, rows, zbuf, ones_v, zdeg, acc_sh, deg_sh, sem = rest
    else:
        acc_out, s_idx, d_idx, rows, zbuf, acc_sh, sem = rest
    c = lax.axis_index("c")
    s = lax.axis_index("s")
    w = s * NC + c

    z16 = jnp.zeros((16,), _f32)
    for i in range(ZR):
        for j in range(D // 16):
            zbuf[i, pl.ds(j * 16, 16)] = z16

    @pl.loop(0, RPT // ZR)
    def _zero(i):
        pltpu.sync_copy(zbuf, acc_sh.at[pl.ds(s * RPT + i * ZR, ZR)])

    if with_deg:
        o16 = jnp.ones((16,), _f32)
        for j in range(CHUNK // 16):
            ones_v[pl.ds(j * 16, 16)] = o16
        for i in range(RPT // 16):
            zdeg[pl.ds(i * 16, 16)] = z16
        pltpu.sync_copy(zdeg, deg_sh.at[pl.ds(s * RPT, RPT)])

    base = w * CH_PER_W
    pltpu.sync_copy(srcs.at[pl.ds(base, CH_PER_W)], s_idx)
    pltpu.sync_copy(dsts.at[pl.ds(base, CH_PER_W)], d_idx)

    plsc.subcore_barrier()

    @pl.loop(0, CH_PER_W)
    def _chunk(j):
        pltpu.async_copy(table.at[s_idx.at[j]], rows, sem).wait()
        pltpu.sync_copy(rows, acc_sh.at[d_idx.at[j]], add=True)
        if with_deg:
            pltpu.sync_copy(ones_v, deg_sh.at[d_idx.at[j]], add=True)

    plsc.subcore_barrier()

    pltpu.sync_copy(acc_sh.at[pl.ds(s * RPT, RPT)],
                    acc_out.at[c, pl.ds(s * RPT, RPT)])
    if with_deg:
        pltpu.sync_copy(deg_sh.at[pl.ds(s * RPT, RPT)],
                        deg_out.at[c, pl.ds(s * RPT, RPT)])


def _make_scatter(D, with_deg):
    mesh = plsc.VectorSubcoreMesh(core_axis_name="c", subcore_axis_name="s",
                                  num_cores=NC, num_subcores=NS)
    out_type = [jax.ShapeDtypeStruct((NC, NP, D), _f32)]
    scratch = [
        pltpu.VMEM((CH_PER_W, CHUNK), jnp.int32),
        pltpu.VMEM((CH_PER_W, CHUNK), jnp.int32),
        pltpu.VMEM((CHUNK, D), _f32),
        pltpu.VMEM((ZR, D), _f32),
    ]
    if with_deg:
        out_type.append(jax.ShapeDtypeStruct((NC, NP), _f32))
        scratch += [pltpu.VMEM((CHUNK,), _f32),
                    pltpu.VMEM((RPT,), _f32)]
    scratch.append(pltpu.VMEM_SHARED((NP, D), _f32))
    if with_deg:
        scratch.append(pltpu.VMEM_SHARED((NP,), _f32))
    scratch.append(pltpu.SemaphoreType.DMA)
    return pl.kernel(
        functools.partial(_scatter_kernel_body, with_deg, D),
        out_type=out_type, mesh=mesh, scratch_types=scratch)


def _dense0_body(acc_ref, deg_ref, x_ref, wl0_ref, wr0_ref, b0_ref,
                 wl1_ref, wr1_ref, b1_ref, hw1_ref, hr1_ref, rec_ref):
    acc = acc_ref[0] + acc_ref[1]
    deg = deg_ref[0] + deg_ref[1]
    rec = 1.0 / jnp.maximum(deg, 1.0)
    agg = acc * rec
    h = agg @ wl0_ref[...] + b0_ref[...] + x_ref[...] @ wr0_ref[...]
    h = jnp.maximum(h, 0.0)
    hw1_ref[...] = h @ wl1_ref[...]
    hr1_ref[...] = h @ wr1_ref[...] + b1_ref[...]
    rec_ref[...] = rec


def _dense1_body(acc_ref, rec_ref, hr1_ref, out_ref):
    out_ref[...] = (acc_ref[0] + acc_ref[1]) * rec_ref[...] + hr1_ref[...]


_BLK = 512
_GRID = NP // _BLK


def _rep(shape):
    return pl.BlockSpec(shape, lambda i: (0,) * len(shape))


def kernel(x, edge_index, W_l0, W_r0, b0, W_l1, W_r1, b1):
    src = edge_index[0]
    dst = edge_index[1]
    npad = E_PAD - E
    pad_src = (jnp.arange(npad, dtype=jnp.int32) % N)
    pad_dst = N + (jnp.arange(npad, dtype=jnp.int32) % (NP - N))
    srcs = jnp.concatenate([src, pad_src]).reshape(E_PAD // CHUNK, CHUNK)
    dsts = jnp.concatenate([dst, pad_dst]).reshape(E_PAD // CHUNK, CHUNK)

    x_pad = jnp.pad(x, ((0, NP - N), (0, 0)))
    wl1p = jnp.pad(W_l1, ((0, 0), (0, 128 - C)))
    wr1p = jnp.pad(W_r1, ((0, 0), (0, 128 - C)))
    b0r = b0.reshape(1, H)
    b1p = jnp.pad(b1, (0, 128 - C)).reshape(1, 128)

    acc0, deg = _make_scatter(H, True)(srcs, dsts, x_pad)
    degr = deg.reshape(NC, NP, 1)

    hw1, hr1, rec = pl.pallas_call(
        _dense0_body,
        grid=(_GRID,),
        in_specs=[
            pl.BlockSpec((NC, _BLK, H), lambda i: (0, i, 0)),
            pl.BlockSpec((NC, _BLK, 1), lambda i: (0, i, 0)),
            pl.BlockSpec((_BLK, F_IN), lambda i: (i, 0)),
            _rep((F_IN, H)), _rep((F_IN, H)), _rep((1, H)),
            _rep((H, 128)), _rep((H, 128)), _rep((1, 128)),
        ],
        out_specs=[
            pl.BlockSpec((_BLK, 128), lambda i: (i, 0)),
            pl.BlockSpec((_BLK, 128), lambda i: (i, 0)),
            pl.BlockSpec((_BLK, 1), lambda i: (i, 0)),
        ],
        out_shape=[
            jax.ShapeDtypeStruct((NP, 128), _f32),
            jax.ShapeDtypeStruct((NP, 128), _f32),
            jax.ShapeDtypeStruct((NP, 1), _f32),
        ],
    )(acc0, degr, x_pad, W_l0, W_r0, b0r, wl1p, wr1p, b1p)

    (acc1,) = _make_scatter(128, False)(srcs, dsts, hw1)

    out = pl.pallas_call(
        _dense1_body,
        grid=(_GRID,),
        in_specs=[
            pl.BlockSpec((NC, _BLK, 128), lambda i: (0, i, 0)),
            pl.BlockSpec((_BLK, 1), lambda i: (i, 0)),
            pl.BlockSpec((_BLK, 128), lambda i: (i, 0)),
        ],
        out_specs=pl.BlockSpec((_BLK, 128), lambda i: (i, 0)),
        out_shape=jax.ShapeDtypeStruct((NP, 128), _f32),
    )(acc1, rec, hr1)

    return out[:N, :C]


# speedup vs baseline: 12.3533x; 1.4400x over previous
import functools

import jax
import jax.numpy as jnp
from jax import lax
from jax.experimental import pallas as pl
from jax.experimental.pallas import tpu as pltpu
from jax.experimental.pallas import tpu_sc as plsc

N = 10000
E = 320000
F_IN = 128
H = 128
C = 40

NC = 2
NS = 16
NW = NC * NS
NP = 10240
RPT = NP // NS
CHUNK = 128
CH_PER_W = 80
E_PAD = NW * CH_PER_W * CHUNK
ZR = 8
NPH = 2
PCH = CH_PER_W // NPH

_f32 = jnp.float32


NBUF = 2


def _scatter_kernel_body(with_deg, D, srcs, dsts, table, *rest):
    if with_deg:
        (acc_out, deg_out, s_idx, d_idx, rows, zbuf, ones_v, zdeg,
         acc_sh, deg_sh) = rest[:10]
        sems = rest[10:]
    else:
        acc_out, s_idx, d_idx, rows, zbuf, acc_sh = rest[:6]
        sems = rest[6:]
    c = lax.axis_index("c")
    s = lax.axis_index("s")
    w = s * NC + c
    base = w * CH_PER_W

    def stage_idx(p):
        pltpu.sync_copy(srcs.at[pl.ds(base + p * PCH, PCH)], s_idx)
        pltpu.sync_copy(dsts.at[pl.ds(base + p * PCH, PCH)], d_idx)

    def prime(_):
        for b in range(NBUF):
            pltpu.async_copy(table.at[s_idx.at[b]], rows.at[b], sems[b])

    def drain_loop(p):
        @pl.loop(0, PCH // NBUF - 1)
        def _group(g):
            for b in range(NBUF):
                j = g * NBUF + b
                pltpu.make_async_copy(table.at[s_idx.at[j]], rows.at[b],
                                      sems[b]).wait()
                pltpu.sync_copy(rows.at[b], acc_sh.at[d_idx.at[j]], add=True)
                if with_deg:
                    pltpu.sync_copy(ones_v, deg_sh.at[d_idx.at[j]], add=True)
                pltpu.async_copy(table.at[s_idx.at[j + NBUF]], rows.at[b],
                                 sems[b])

        for b in range(NBUF):
            j = PCH - NBUF + b
            pltpu.make_async_copy(table.at[s_idx.at[j]], rows.at[b],
                                  sems[b]).wait()
            pltpu.sync_copy(rows.at[b], acc_sh.at[d_idx.at[j]], add=True)
            if with_deg:
                pltpu.sync_copy(ones_v, deg_sh.at[d_idx.at[j]], add=True)

    stage_idx(0)
    prime(0)

    z16 = jnp.zeros((16,), _f32)
    for i in range(ZR):
        for j in range(D // 16):
            zbuf[i, pl.ds(j * 16, 16)] = z16

    @pl.loop(0, RPT // ZR)
    def _zero(i):
        pltpu.sync_copy(zbuf, acc_sh.at[pl.ds(s * RPT + i * ZR, ZR)])

    if with_deg:
        o16 = jnp.ones((16,), _f32)
        for j in range(CHUNK // 16):
            ones_v[pl.ds(j * 16, 16)] = o16
        for i in range(RPT // 16):
            zdeg[pl.ds(i * 16, 16)] = z16
        pltpu.sync_copy(zdeg, deg_sh.at[pl.ds(s * RPT, RPT)])

    plsc.subcore_barrier()

    drain_loop(0)
    for p in range(1, NPH):
        stage_idx(p)
        prime(p)
        drain_loop(p)

    plsc.subcore_barrier()

    pltpu.sync_copy(acc_sh.at[pl.ds(s * RPT, RPT)],
                    acc_out.at[c, pl.ds(s * RPT, RPT)])
    if with_deg:
        pltpu.sync_copy(deg_sh.at[pl.ds(s * RPT, RPT)],
                        deg_out.at[c, pl.ds(s * RPT, RPT)])


def _make_scatter(D, with_deg):
    mesh = plsc.VectorSubcoreMesh(core_axis_name="c", subcore_axis_name="s",
                                  num_cores=NC, num_subcores=NS)
    out_type = [jax.ShapeDtypeStruct((NC, NP, D), _f32)]
    scratch = [
        pltpu.VMEM((PCH, CHUNK), jnp.int32),
        pltpu.VMEM((PCH, CHUNK), jnp.int32),
        pltpu.VMEM((NBUF, CHUNK, D), _f32),
        pltpu.VMEM((ZR, D), _f32),
    ]
    if with_deg:
        out_type.append(jax.ShapeDtypeStruct((NC, NP), _f32))
        scratch += [pltpu.VMEM((CHUNK,), _f32),
                    pltpu.VMEM((RPT,), _f32)]
    scratch.append(pltpu.VMEM_SHARED((NP, D), _f32))
    if with_deg:
        scratch.append(pltpu.VMEM_SHARED((NP,), _f32))
    scratch.extend([pltpu.SemaphoreType.DMA] * NBUF)
    return pl.kernel(
        functools.partial(_scatter_kernel_body, with_deg, D),
        out_type=out_type, mesh=mesh, scratch_types=scratch)


def _dense0_body(acc_ref, deg_ref, x_ref, wl0_ref, wr0_ref, b0_ref,
                 wl1_ref, wr1_ref, b1_ref, hw1_ref, hr1_ref, rec_ref):
    acc = acc_ref[0] + acc_ref[1]
    deg = deg_ref[0] + deg_ref[1]
    rec = 1.0 / jnp.maximum(deg, 1.0)
    agg = acc * rec
    h = agg @ wl0_ref[...] + b0_ref[...] + x_ref[...] @ wr0_ref[...]
    h = jnp.maximum(h, 0.0)
    hw1_ref[...] = h @ wl1_ref[...]
    hr1_ref[...] = h @ wr1_ref[...] + b1_ref[...]
    rec_ref[...] = rec


def _dense1_body(acc_ref, rec_ref, hr1_ref, out_ref):
    out_ref[...] = (acc_ref[0] + acc_ref[1]) * rec_ref[...] + hr1_ref[...]


_BLK = 512
_GRID = NP // _BLK


def _rep(shape):
    return pl.BlockSpec(shape, lambda i: (0,) * len(shape))


def kernel(x, edge_index, W_l0, W_r0, b0, W_l1, W_r1, b1):
    src = edge_index[0]
    dst = edge_index[1]
    npad = E_PAD - E
    pad_src = (jnp.arange(npad, dtype=jnp.int32) % N)
    pad_dst = N + (jnp.arange(npad, dtype=jnp.int32) % (NP - N))
    srcs = jnp.concatenate([src, pad_src]).reshape(E_PAD // CHUNK, CHUNK)
    dsts = jnp.concatenate([dst, pad_dst]).reshape(E_PAD // CHUNK, CHUNK)

    x_pad = jnp.pad(x, ((0, NP - N), (0, 0)))
    wl1p = jnp.pad(W_l1, ((0, 0), (0, 128 - C)))
    wr1p = jnp.pad(W_r1, ((0, 0), (0, 128 - C)))
    b0r = b0.reshape(1, H)
    b1p = jnp.pad(b1, (0, 128 - C)).reshape(1, 128)

    acc0, deg = _make_scatter(H, True)(srcs, dsts, x_pad)
    degr = deg.reshape(NC, NP, 1)

    hw1, hr1, rec = pl.pallas_call(
        _dense0_body,
        grid=(_GRID,),
        in_specs=[
            pl.BlockSpec((NC, _BLK, H), lambda i: (0, i, 0)),
            pl.BlockSpec((NC, _BLK, 1), lambda i: (0, i, 0)),
            pl.BlockSpec((_BLK, F_IN), lambda i: (i, 0)),
            _rep((F_IN, H)), _rep((F_IN, H)), _rep((1, H)),
            _rep((H, 128)), _rep((H, 128)), _rep((1, 128)),
        ],
        out_specs=[
            pl.BlockSpec((_BLK, 128), lambda i: (i, 0)),
            pl.BlockSpec((_BLK, 128), lambda i: (i, 0)),
            pl.BlockSpec((_BLK, 1), lambda i: (i, 0)),
        ],
        out_shape=[
            jax.ShapeDtypeStruct((NP, 128), _f32),
            jax.ShapeDtypeStruct((NP, 128), _f32),
            jax.ShapeDtypeStruct((NP, 1), _f32),
        ],
    )(acc0, degr, x_pad, W_l0, W_r0, b0r, wl1p, wr1p, b1p)

    (acc1,) = _make_scatter(128, False)(srcs, dsts, hw1)

    out = pl.pallas_call(
        _dense1_body,
        grid=(_GRID,),
        in_specs=[
            pl.BlockSpec((NC, _BLK, 128), lambda i: (0, i, 0)),
            pl.BlockSpec((_BLK, 1), lambda i: (i, 0)),
            pl.BlockSpec((_BLK, 128), lambda i: (i, 0)),
        ],
        out_specs=pl.BlockSpec((_BLK, 128), lambda i: (i, 0)),
        out_shape=jax.ShapeDtypeStruct((NP, 128), _f32),
    )(acc1, rec, hr1)

    return out[:N, :C]
